# Initial kernel scaffold; baseline (speedup 1.0000x reference)
import jax
import jax.numpy as jnp
from jax.experimental import pallas as pl


def kernel(E0, users, pos_movies, neg_movies, user_index, movie_index):
    raise NotImplementedError("write your pallas kernel here")


# speedup vs baseline: 5.9371x; 5.9371x over previous
import functools

import jax
import jax.numpy as jnp
from jax import lax
from jax.experimental import pallas as pl
from jax.experimental.pallas import tpu as pltpu
from jax.experimental.pallas import tpu_sc as plsc

NU = 50000
NM = 50000
NT = NU + NM
NE = 800000
DIM = 64
DH = 32
NLAYERS = 3
NC = 2
NS = 16
CHUNK = 80
EPT = NE // NS
NCHUNK = EPT // CHUNK
FCH = 400
NFCH = NU // FCH

_mesh = plsc.VectorSubcoreMesh(core_axis_name="c", subcore_axis_name="s",
                               num_cores=NC, num_subcores=NS)
_sc_params = pltpu.CompilerParams(use_tc_tiling_on_sc=False)


def _zero_fill(buf, nrows):
    z = jnp.zeros((16,), jnp.float32)

    def body(i, carry):
        buf[i, pl.ds(0, 16)] = z
        return carry

    lax.fori_loop(0, nrows, body, 0)


def _zero_fill32(buf, nrows):
    z = jnp.zeros((16,), jnp.float32)

    def body(i, carry):
        buf[i, pl.ds(0, 16)] = z
        buf[i, pl.ds(16, 16)] = z
        return carry

    lax.fori_loop(0, nrows, body, 0)


@functools.partial(
    pl.kernel,
    out_type=jax.ShapeDtypeStruct((NT, 16), jnp.float32),
    mesh=_mesh,
    compiler_params=_sc_params,
    scratch_types=[
        pltpu.VMEM((CHUNK,), jnp.int32),
        pltpu.VMEM((CHUNK, 16), jnp.float32),
        pltpu.VMEM((FCH, 16), jnp.float32),
        pltpu.VMEM_SHARED((NU, 16), jnp.float32),
    ],
)
def _degree_kernel(uidx_hbm, midx_hbm, deg_hbm, idxb, onesb, stage, acc):
    c = lax.axis_index("c")
    s = lax.axis_index("s")

    one = jnp.ones((16,), jnp.float32)

    def ones_body(i, carry):
        onesb[i, pl.ds(0, 16)] = one
        return carry

    lax.fori_loop(0, CHUNK, ones_body, 0)
    _zero_fill(stage, FCH)

    nfl = 7 + (s < 13).astype(jnp.int32)

    def zchunk(i, carry):
        pltpu.sync_copy(stage, acc.at[pl.ds((s + i * NS) * FCH, FCH)])
        return carry

    lax.fori_loop(0, nfl, zchunk, 0)
    plsc.subcore_barrier()

    def run_side(side_hbm):
        base = s * EPT

        def chunk(j, carry):
            pltpu.sync_copy(side_hbm.at[pl.ds(base + j * CHUNK, CHUNK)], idxb)
            pltpu.sync_copy(onesb, acc.at[idxb], add=True)
            return carry

        lax.fori_loop(0, NCHUNK, chunk, 0)

    @pl.when(c == 0)
    def _():
        run_side(uidx_hbm)

    @pl.when(c == 1)
    def _():
        run_side(midx_hbm)

    plsc.subcore_barrier()
    def fchunk(i, carry):
        r = (s + i * NS) * FCH
        pltpu.sync_copy(acc.at[pl.ds(r, FCH)], stage)
        pltpu.sync_copy(stage, deg_hbm.at[pl.ds(c * NU + r, FCH)])
        return carry

    lax.fori_loop(0, nfl, fchunk, 0)


@functools.partial(
    pl.kernel,
    out_type=(jax.ShapeDtypeStruct((NT, DH), jnp.float32),
              jax.ShapeDtypeStruct((NT, DH), jnp.float32)),
    mesh=_mesh,
    compiler_params=_sc_params,
    scratch_types=[
        pltpu.VMEM((CHUNK,), jnp.int32),
        pltpu.VMEM((CHUNK,), jnp.int32),
        pltpu.VMEM((CHUNK, DH), jnp.float32),
        pltpu.VMEM((FCH, DH), jnp.float32),
        pltpu.VMEM((FCH, DH), jnp.float32),
        pltpu.VMEM_SHARED((NU, DH), jnp.float32),
        pltpu.SemaphoreType.DMA,
    ],
)
def _propagate_kernel(slo_hbm, shi_hbm, uidx_hbm, midx_hbm, midxp_hbm,
                      tlo_hbm, thi_hbm,
                      sidx, didx, rows, zbuf, fbuf, acc, sem):
    c = lax.axis_index("c")
    s = lax.axis_index("s")

    _zero_fill32(zbuf, FCH)

    nfl = 7 + (s < 13).astype(jnp.int32)

    def zero_acc():
        def zchunk(i, carry):
            pltpu.sync_copy(zbuf, acc.at[pl.ds((s + i * NS) * FCH, FCH)])
            return carry

        lax.fori_loop(0, nfl, zchunk, 0)

    def phase(src_hbm, dst_hbm, table_hbm):
        base = s * EPT

        def chunk(j, carry):
            pltpu.sync_copy(src_hbm.at[pl.ds(base + j * CHUNK, CHUNK)], sidx)
            pltpu.sync_copy(dst_hbm.at[pl.ds(base + j * CHUNK, CHUNK)], didx)
            pltpu.async_copy(table_hbm.at[sidx], rows, sem).wait()
            pltpu.sync_copy(rows, acc.at[didx], add=True)
            return carry

        lax.fori_loop(0, NCHUNK, chunk, 0)

    def flush(out_hbm, row_off):
        def fchunk(i, carry):
            r = (s + i * NS) * FCH
            pltpu.sync_copy(acc.at[pl.ds(r, FCH)], fbuf)
            pltpu.sync_copy(fbuf, out_hbm.at[pl.ds(row_off + r, FCH)])
            return carry

        lax.fori_loop(0, nfl, fchunk, 0)

    def run(table_hbm, out_hbm):
        zero_acc()
        plsc.subcore_barrier()
        phase(midxp_hbm, uidx_hbm, table_hbm)
        plsc.subcore_barrier()
        flush(out_hbm, 0)
        zero_acc()
        plsc.subcore_barrier()
        phase(uidx_hbm, midx_hbm, table_hbm)
        plsc.subcore_barrier()
        flush(out_hbm, NU)

    @pl.when(c == 0)
    def _():
        run(slo_hbm, tlo_hbm)

    @pl.when(c == 1)
    def _():
        run(shi_hbm, thi_hbm)


NB_OUT = 4096 + 4096 + 4 * 4096
ROWS_PER_TILE = NB_OUT // (NC * NS)
GCH = 128

@functools.partial(
    pl.kernel,
    out_type=jax.ShapeDtypeStruct((NB_OUT, DIM), jnp.float32),
    mesh=_mesh,
    compiler_params=_sc_params,
    scratch_types=[
        pltpu.VMEM((GCH,), jnp.int32),
        pltpu.VMEM((GCH, DIM), jnp.float32),
        pltpu.SemaphoreType.DMA,
    ],
)
def _lookup_kernel(ef_hbm, idx_hbm, out_hbm, gidx, grows, sem):
    c = lax.axis_index("c")
    s = lax.axis_index("s")
    wid = s * NC + c
    for j in range(ROWS_PER_TILE // GCH):
        base = wid * ROWS_PER_TILE + j * GCH
        pltpu.sync_copy(idx_hbm.at[pl.ds(base, GCH)], gidx)
        pltpu.async_copy(ef_hbm.at[gidx], grows, sem).wait()
        pltpu.sync_copy(grows, out_hbm.at[pl.ds(base, GCH)])


_TCROWS = 800
_TCGRID = NT // _TCROWS


def _prep_body(deg_ref, e0_ref, slo_ref, shi_ref):
    d = lax.rsqrt(deg_ref[:, 0:1] + 1e-9)
    srow = e0_ref[...] * d
    slo_ref[...] = srow[:, :DH]
    shi_ref[...] = srow[:, DH:]


def _tc_prep(deg16, e0):
    return pl.pallas_call(
        _prep_body,
        grid=(_TCGRID,),
        in_specs=[pl.BlockSpec((_TCROWS, 16), lambda i: (i, 0)),
                  pl.BlockSpec((_TCROWS, DIM), lambda i: (i, 0))],
        out_specs=[pl.BlockSpec((_TCROWS, DH), lambda i: (i, 0)),
                   pl.BlockSpec((_TCROWS, DH), lambda i: (i, 0))],
        out_shape=[jax.ShapeDtypeStruct((NT, DH), jnp.float32),
                   jax.ShapeDtypeStruct((NT, DH), jnp.float32)],
    )(deg16, e0)


def _layer_body(tlo_ref, thi_ref, deg_ref, esum_ref,
                slo_ref, shi_ref, esumo_ref):
    d = lax.rsqrt(deg_ref[:, 0:1] + 1e-9)
    t = jnp.concatenate([tlo_ref[...], thi_ref[...]], axis=1)
    e = d * t
    esumo_ref[...] = esum_ref[...] + e
    srow = d * e
    slo_ref[...] = srow[:, :DH]
    shi_ref[...] = srow[:, DH:]


def _tc_layer(tlo, thi, deg16, esum):
    return pl.pallas_call(
        _layer_body,
        grid=(_TCGRID,),
        in_specs=[pl.BlockSpec((_TCROWS, DH), lambda i: (i, 0)),
                  pl.BlockSpec((_TCROWS, DH), lambda i: (i, 0)),
                  pl.BlockSpec((_TCROWS, 16), lambda i: (i, 0)),
                  pl.BlockSpec((_TCROWS, DIM), lambda i: (i, 0))],
        out_specs=[pl.BlockSpec((_TCROWS, DH), lambda i: (i, 0)),
                   pl.BlockSpec((_TCROWS, DH), lambda i: (i, 0)),
                   pl.BlockSpec((_TCROWS, DIM), lambda i: (i, 0))],
        out_shape=[jax.ShapeDtypeStruct((NT, DH), jnp.float32),
                   jax.ShapeDtypeStruct((NT, DH), jnp.float32),
                   jax.ShapeDtypeStruct((NT, DIM), jnp.float32)],
    )(tlo, thi, deg16, esum)


def _final_body(tlo_ref, thi_ref, deg_ref, esum_ref, ef_ref):
    d = lax.rsqrt(deg_ref[:, 0:1] + 1e-9)
    t = jnp.concatenate([tlo_ref[...], thi_ref[...]], axis=1)
    ef_ref[...] = (esum_ref[...] + d * t) * 0.25


def _tc_final(tlo, thi, deg16, esum):
    return pl.pallas_call(
        _final_body,
        grid=(_TCGRID,),
        in_specs=[pl.BlockSpec((_TCROWS, DH), lambda i: (i, 0)),
                  pl.BlockSpec((_TCROWS, DH), lambda i: (i, 0)),
                  pl.BlockSpec((_TCROWS, 16), lambda i: (i, 0)),
                  pl.BlockSpec((_TCROWS, DIM), lambda i: (i, 0))],
        out_specs=pl.BlockSpec((_TCROWS, DIM), lambda i: (i, 0)),
        out_shape=jax.ShapeDtypeStruct((NT, DIM), jnp.float32),
    )(tlo, thi, deg16, esum)


def kernel(E0, users, pos_movies, neg_movies, user_index, movie_index):
    movie_plus = movie_index + NU
    deg16 = _degree_kernel(user_index, movie_index)
    slo, shi = _tc_prep(deg16, E0)
    esum = E0
    for _ in range(NLAYERS - 1):
        tlo, thi = _propagate_kernel(slo, shi, user_index, movie_index,
                                     movie_plus)
        slo, shi, esum = _tc_layer(tlo, thi, deg16, esum)
    tlo, thi = _propagate_kernel(slo, shi, user_index, movie_index, movie_plus)
    ef = _tc_final(tlo, thi, deg16, esum)

    all_idx = jnp.concatenate([users, pos_movies + NU, neg_movies + NU])
    out_rows = _lookup_kernel(ef, all_idx)
    usr = out_rows[:4096]
    pos = out_rows[4096:8192]
    neg = out_rows[8192:]
    return (usr, pos, neg)


# speedup vs baseline: 15.0212x; 2.5301x over previous
import functools

import jax
import jax.numpy as jnp
from jax import lax
from jax.experimental import pallas as pl
from jax.experimental.pallas import tpu as pltpu
from jax.experimental.pallas import tpu_sc as plsc

NU = 50000
NM = 50000
NT = NU + NM
NE = 800000
DIM = 64
DH = 32
NLAYERS = 3
NC = 2
NS = 16
CB = 128
EPT = NE // NS
NCH = EPT // CB
TAIL = EPT - NCH * CB
FCH = 200

_mesh = plsc.VectorSubcoreMesh(core_axis_name="c", subcore_axis_name="s",
                               num_cores=NC, num_subcores=NS)
_sc_params = pltpu.CompilerParams(use_tc_tiling_on_sc=False)


def _zero_fill(buf, nrows, ncols):
    z = jnp.zeros((16,), jnp.float32)

    def body(i, carry):
        for c0 in range(0, ncols, 16):
            buf[i, pl.ds(c0, 16)] = z
        return carry

    lax.fori_loop(0, nrows, body, 0)


@functools.partial(
    pl.kernel,
    out_type=jax.ShapeDtypeStruct((NT, 16), jnp.float32),
    mesh=_mesh,
    compiler_params=_sc_params,
    scratch_types=[
        pltpu.VMEM((CB,), jnp.int32),
        pltpu.VMEM((CB,), jnp.int32),
        pltpu.VMEM((TAIL,), jnp.int32),
        pltpu.VMEM((CB, 16), jnp.float32),
        pltpu.VMEM((FCH, 16), jnp.float32),
        pltpu.VMEM_SHARED((NU, 16), jnp.float32),
        pltpu.SemaphoreType.DMA,
        pltpu.SemaphoreType.DMA,
    ],
)
def _degree_kernel(uidx_hbm, midx_hbm, deg_hbm,
                   idx0, idx1, tidx, onesb, stage, acc, sem0, sem1):
    c = lax.axis_index("c")
    s = lax.axis_index("s")

    one = jnp.ones((16,), jnp.float32)

    def ones_body(i, carry):
        onesb[i, pl.ds(0, 16)] = one
        return carry

    lax.fori_loop(0, CB, ones_body, 0)
    _zero_fill(stage, FCH, 16)

    nfl = 15 + (s < 10).astype(jnp.int32)

    def zchunk(i, carry):
        pltpu.sync_copy(stage, acc.at[pl.ds((s + i * NS) * FCH, FCH)])
        return carry

    lax.fori_loop(0, nfl, zchunk, 0)
    plsc.subcore_barrier()

    def run_side(side_hbm):
        base = s * EPT

        def ioff(j):
            return jnp.minimum(base + j * CB, NE - CB)

        pltpu.async_copy(side_hbm.at[pl.ds(ioff(0), CB)], idx0, sem0)
        pltpu.async_copy(side_hbm.at[pl.ds(ioff(1), CB)], idx1, sem1)

        def wait_i(buf, sem):
            pltpu.make_async_copy(side_hbm.at[pl.ds(0, CB)], buf, sem).wait()

        def pair(t, carry):
            j0 = 2 * t
            wait_i(idx0, sem0)
            pltpu.sync_copy(onesb, acc.at[idx0], add=True)
            pltpu.async_copy(side_hbm.at[pl.ds(ioff(j0 + 2), CB)], idx0, sem0)
            wait_i(idx1, sem1)
            pltpu.sync_copy(onesb, acc.at[idx1], add=True)
            pltpu.async_copy(side_hbm.at[pl.ds(ioff(j0 + 3), CB)], idx1, sem1)
            return carry

        lax.fori_loop(0, NCH // 2, pair, 0)
        wait_i(idx0, sem0)
        wait_i(idx1, sem1)
        pltpu.sync_copy(side_hbm.at[pl.ds(base + NCH * CB, TAIL)], tidx)
        pltpu.sync_copy(onesb.at[pl.ds(0, TAIL)], acc.at[tidx], add=True)

    @pl.when(c == 0)
    def _():
        run_side(uidx_hbm)

    @pl.when(c == 1)
    def _():
        run_side(midx_hbm)

    plsc.subcore_barrier()
    def fchunk(i, carry):
        r = (s + i * NS) * FCH
        pltpu.sync_copy(acc.at[pl.ds(r, FCH)], stage)
        pltpu.sync_copy(stage, deg_hbm.at[pl.ds(c * NU + r, FCH)])
        return carry

    lax.fori_loop(0, nfl, fchunk, 0)


@functools.partial(
    pl.kernel,
    out_type=(jax.ShapeDtypeStruct((NT, DH), jnp.float32),
              jax.ShapeDtypeStruct((NT, DH), jnp.float32)),
    mesh=_mesh,
    compiler_params=_sc_params,
    scratch_types=[
        pltpu.VMEM((CB,), jnp.int32),
        pltpu.VMEM((CB,), jnp.int32),
        pltpu.VMEM((CB,), jnp.int32),
        pltpu.VMEM((CB,), jnp.int32),
        pltpu.VMEM((CB, DH), jnp.float32),
        pltpu.VMEM((CB, DH), jnp.float32),
        pltpu.VMEM((TAIL,), jnp.int32),
        pltpu.VMEM((TAIL,), jnp.int32),
        pltpu.VMEM((TAIL, DH), jnp.float32),
        pltpu.VMEM((FCH, DH), jnp.float32),
        pltpu.VMEM((FCH, DH), jnp.float32),
        pltpu.VMEM_SHARED((NU, DH), jnp.float32),
        pltpu.SemaphoreType.DMA,
        pltpu.SemaphoreType.DMA,
        pltpu.SemaphoreType.DMA,
        pltpu.SemaphoreType.DMA,
    ],
)
def _propagate_kernel(slo_hbm, shi_hbm, uidx_hbm, midx_hbm, midxp_hbm,
                      tlo_hbm, thi_hbm,
                      sidx0, sidx1, didx0, didx1, rows0, rows1,
                      tsidx, tdidx, trows, zbuf, fbuf, acc,
                      semi0, semi1, semg0, semg1):
    c = lax.axis_index("c")
    s = lax.axis_index("s")

    _zero_fill(zbuf, FCH, DH)

    nfl = 15 + (s < 10).astype(jnp.int32)

    def zero_acc():
        def zchunk(i, carry):
            pltpu.sync_copy(zbuf, acc.at[pl.ds((s + i * NS) * FCH, FCH)])
            return carry

        lax.fori_loop(0, nfl, zchunk, 0)

    def phase(src_hbm, dst_hbm, table_hbm):
        base = s * EPT

        def ioff(j):
            return jnp.minimum(base + j * CB, NE - CB)

        def issue_idx(j, si, di, sem):
            off = ioff(j)
            pltpu.async_copy(src_hbm.at[pl.ds(off, CB)], si, sem)
            pltpu.async_copy(dst_hbm.at[pl.ds(off, CB)], di, sem)

        def wait_idx(si, di, sem):
            pltpu.make_async_copy(src_hbm.at[pl.ds(0, CB)], si, sem).wait()
            pltpu.make_async_copy(dst_hbm.at[pl.ds(0, CB)], di, sem).wait()

        def wait_gather(si, rw, sem):
            pltpu.make_async_copy(table_hbm.at[si], rw, sem).wait()

        issue_idx(0, sidx0, didx0, semi0)
        issue_idx(1, sidx1, didx1, semi1)
        wait_idx(sidx0, didx0, semi0)
        pltpu.async_copy(table_hbm.at[sidx0], rows0, semg0)

        def pair(t, carry):
            j0 = 2 * t
            wait_idx(sidx1, didx1, semi1)
            pltpu.async_copy(table_hbm.at[sidx1], rows1, semg1)
            wait_gather(sidx0, rows0, semg0)
            pltpu.sync_copy(rows0, acc.at[didx0], add=True)
            issue_idx(j0 + 2, sidx0, didx0, semi0)
            wait_idx(sidx0, didx0, semi0)
            pltpu.async_copy(table_hbm.at[sidx0], rows0, semg0)
            wait_gather(sidx1, rows1, semg1)
            pltpu.sync_copy(rows1, acc.at[didx1], add=True)
            issue_idx(j0 + 3, sidx1, didx1, semi1)
            return carry

        lax.fori_loop(0, NCH // 2, pair, 0)
        wait_gather(sidx0, rows0, semg0)
        wait_idx(sidx1, didx1, semi1)
        toff = base + NCH * CB
        pltpu.sync_copy(src_hbm.at[pl.ds(toff, TAIL)], tsidx)
        pltpu.sync_copy(dst_hbm.at[pl.ds(toff, TAIL)], tdidx)
        pltpu.async_copy(table_hbm.at[tsidx], trows, semg0).wait()
        pltpu.sync_copy(trows, acc.at[tdidx], add=True)

    def flush(out_hbm, row_off):
        def fchunk(i, carry):
            r = (s + i * NS) * FCH
            pltpu.sync_copy(acc.at[pl.ds(r, FCH)], fbuf)
            pltpu.sync_copy(fbuf, out_hbm.at[pl.ds(row_off + r, FCH)])
            return carry

        lax.fori_loop(0, nfl, fchunk, 0)

    def run(table_hbm, out_hbm):
        zero_acc()
        plsc.subcore_barrier()
        phase(midxp_hbm, uidx_hbm, table_hbm)
        plsc.subcore_barrier()
        flush(out_hbm, 0)
        zero_acc()
        plsc.subcore_barrier()
        phase(uidx_hbm, midx_hbm, table_hbm)
        plsc.subcore_barrier()
        flush(out_hbm, NU)

    @pl.when(c == 0)
    def _():
        run(slo_hbm, tlo_hbm)

    @pl.when(c == 1)
    def _():
        run(shi_hbm, thi_hbm)


NB_OUT = 4096 + 4096 + 4 * 4096
ROWS_PER_TILE = NB_OUT // (NC * NS)
GCH = 128

@functools.partial(
    pl.kernel,
    out_type=jax.ShapeDtypeStruct((NB_OUT, DIM), jnp.float32),
    mesh=_mesh,
    compiler_params=_sc_params,
    scratch_types=[
        pltpu.VMEM((GCH,), jnp.int32),
        pltpu.VMEM((GCH, DIM), jnp.float32),
        pltpu.SemaphoreType.DMA,
    ],
)
def _lookup_kernel(ef_hbm, idx_hbm, out_hbm, gidx, grows, sem):
    c = lax.axis_index("c")
    s = lax.axis_index("s")
    wid = s * NC + c
    for j in range(ROWS_PER_TILE // GCH):
        base = wid * ROWS_PER_TILE + j * GCH
        pltpu.sync_copy(idx_hbm.at[pl.ds(base, GCH)], gidx)
        pltpu.async_copy(ef_hbm.at[gidx], grows, sem).wait()
        pltpu.sync_copy(grows, out_hbm.at[pl.ds(base, GCH)])


_TCROWS = 800
_TCGRID = NT // _TCROWS


def _prep_body(deg_ref, e0_ref, slo_ref, shi_ref):
    d = lax.rsqrt(deg_ref[:, 0:1] + 1e-9)
    srow = e0_ref[...] * d
    slo_ref[...] = srow[:, :DH]
    shi_ref[...] = srow[:, DH:]


def _tc_prep(deg16, e0):
    return pl.pallas_call(
        _prep_body,
        grid=(_TCGRID,),
        in_specs=[pl.BlockSpec((_TCROWS, 16), lambda i: (i, 0)),
                  pl.BlockSpec((_TCROWS, DIM), lambda i: (i, 0))],
        out_specs=[pl.BlockSpec((_TCROWS, DH), lambda i: (i, 0)),
                   pl.BlockSpec((_TCROWS, DH), lambda i: (i, 0))],
        out_shape=[jax.ShapeDtypeStruct((NT, DH), jnp.float32),
                   jax.ShapeDtypeStruct((NT, DH), jnp.float32)],
    )(deg16, e0)


def _layer_body(tlo_ref, thi_ref, deg_ref, esum_ref,
                slo_ref, shi_ref, esumo_ref):
    d = lax.rsqrt(deg_ref[:, 0:1] + 1e-9)
    t = jnp.concatenate([tlo_ref[...], thi_ref[...]], axis=1)
    e = d * t
    esumo_ref[...] = esum_ref[...] + e
    srow = d * e
    slo_ref[...] = srow[:, :DH]
    shi_ref[...] = srow[:, DH:]


def _tc_layer(tlo, thi, deg16, esum):
    return pl.pallas_call(
        _layer_body,
        grid=(_TCGRID,),
        in_specs=[pl.BlockSpec((_TCROWS, DH), lambda i: (i, 0)),
                  pl.BlockSpec((_TCROWS, DH), lambda i: (i, 0)),
                  pl.BlockSpec((_TCROWS, 16), lambda i: (i, 0)),
                  pl.BlockSpec((_TCROWS, DIM), lambda i: (i, 0))],
        out_specs=[pl.BlockSpec((_TCROWS, DH), lambda i: (i, 0)),
                   pl.BlockSpec((_TCROWS, DH), lambda i: (i, 0)),
                   pl.BlockSpec((_TCROWS, DIM), lambda i: (i, 0))],
        out_shape=[jax.ShapeDtypeStruct((NT, DH), jnp.float32),
                   jax.ShapeDtypeStruct((NT, DH), jnp.float32),
                   jax.ShapeDtypeStruct((NT, DIM), jnp.float32)],
    )(tlo, thi, deg16, esum)


def _final_body(tlo_ref, thi_ref, deg_ref, esum_ref, ef_ref):
    d = lax.rsqrt(deg_ref[:, 0:1] + 1e-9)
    t = jnp.concatenate([tlo_ref[...], thi_ref[...]], axis=1)
    ef_ref[...] = (esum_ref[...] + d * t) * 0.25


def _tc_final(tlo, thi, deg16, esum):
    return pl.pallas_call(
        _final_body,
        grid=(_TCGRID,),
        in_specs=[pl.BlockSpec((_TCROWS, DH), lambda i: (i, 0)),
                  pl.BlockSpec((_TCROWS, DH), lambda i: (i, 0)),
                  pl.BlockSpec((_TCROWS, 16), lambda i: (i, 0)),
                  pl.BlockSpec((_TCROWS, DIM), lambda i: (i, 0))],
        out_specs=pl.BlockSpec((_TCROWS, DIM), lambda i: (i, 0)),
        out_shape=jax.ShapeDtypeStruct((NT, DIM), jnp.float32),
    )(tlo, thi, deg16, esum)


def kernel(E0, users, pos_movies, neg_movies, user_index, movie_index):
    movie_plus = movie_index + NU
    deg16 = _degree_kernel(user_index, movie_index)
    slo, shi = _tc_prep(deg16, E0)
    esum = E0
    for _ in range(NLAYERS - 1):
        tlo, thi = _propagate_kernel(slo, shi, user_index, movie_index,
                                     movie_plus)
        slo, shi, esum = _tc_layer(tlo, thi, deg16, esum)
    tlo, thi = _propagate_kernel(slo, shi, user_index, movie_index, movie_plus)
    ef = _tc_final(tlo, thi, deg16, esum)

    all_idx = jnp.concatenate([users, pos_movies + NU, neg_movies + NU])
    out_rows = _lookup_kernel(ef, all_idx)
    usr = out_rows[:4096]
    pos = out_rows[4096:8192]
    neg = out_rows[8192:]
    return (usr, pos, neg)


# speedup vs baseline: 17.8886x; 1.1909x over previous
import functools

import jax
import jax.numpy as jnp
from jax import lax
from jax.experimental import pallas as pl
from jax.experimental.pallas import tpu as pltpu
from jax.experimental.pallas import tpu_sc as plsc

NU = 50000
NM = 50000
NT = NU + NM
NE = 800000
DIM = 64
DH = 32
NC = 2
NS = 16
CB = 128
EPT = NE // NS
NCH = EPT // CB
TAIL = EPT - NCH * CB
DFCH = 200
FCH = 80
NFC = NU // FCH
NPC = NT // FCH
NB_OUT = 4096 + 4096 + 4 * 4096
RPT = NB_OUT // NS
GCH = 64
NGC = RPT // GCH

_mesh = plsc.VectorSubcoreMesh(core_axis_name="c", subcore_axis_name="s",
                               num_cores=NC, num_subcores=NS)
_sc_params = pltpu.CompilerParams(use_tc_tiling_on_sc=False)


def _zero_fill(buf, nrows, ncols):
    z = jnp.zeros((16,), jnp.float32)

    def body(i, carry):
        for c0 in range(0, ncols, 16):
            buf[i, pl.ds(c0, 16)] = z
        return carry

    lax.fori_loop(0, nrows, body, 0)


def _rsqrt16(x):
    magic = jnp.full((16,), 0x5F3759DF, jnp.int32)
    one = jnp.full((16,), 1, jnp.int32)
    c15 = jnp.full((16,), 1.5, jnp.float32)
    xi = lax.bitcast_convert_type(x, jnp.int32)
    y = lax.bitcast_convert_type(magic - lax.shift_right_logical(xi, one),
                                 jnp.float32)
    nhx = x * jnp.full((16,), -0.5, jnp.float32)
    for _ in range(4):
        y = y * (c15 + nhx * y * y)
    return y


@functools.partial(
    pl.kernel,
    out_type=jax.ShapeDtypeStruct((NT, 16), jnp.float32),
    mesh=_mesh,
    compiler_params=_sc_params,
    scratch_types=[
        pltpu.VMEM((CB,), jnp.int32),
        pltpu.VMEM((CB,), jnp.int32),
        pltpu.VMEM((TAIL,), jnp.int32),
        pltpu.VMEM((CB, 16), jnp.float32),
        pltpu.VMEM((DFCH, 16), jnp.float32),
        pltpu.VMEM((DFCH, 16), jnp.float32),
        pltpu.VMEM_SHARED((NU, 16), jnp.float32),
        pltpu.SemaphoreType.DMA,
        pltpu.SemaphoreType.DMA,
    ],
)
def _degree_kernel(uidx_hbm, midx_hbm, dtab_hbm,
                   idx0, idx1, tidx, onesb, stage, stage2, acc, sem0, sem1):
    c = lax.axis_index("c")
    s = lax.axis_index("s")

    one = jnp.ones((16,), jnp.float32)

    def ones_body(i, carry):
        onesb[i, pl.ds(0, 16)] = one
        return carry

    lax.fori_loop(0, CB, ones_body, 0)
    _zero_fill(stage, DFCH, 16)

    nfl = 15 + (s < 10).astype(jnp.int32)

    def zchunk(i, carry):
        pltpu.sync_copy(stage, acc.at[pl.ds((s + i * NS) * DFCH, DFCH)])
        return carry

    lax.fori_loop(0, nfl, zchunk, 0)
    plsc.subcore_barrier()

    def run_side(side_hbm):
        base = s * EPT

        def ioff(j):
            return jnp.minimum(base + j * CB, NE - CB)

        pltpu.async_copy(side_hbm.at[pl.ds(ioff(0), CB)], idx0, sem0)
        pltpu.async_copy(side_hbm.at[pl.ds(ioff(1), CB)], idx1, sem1)

        def wait_i(buf, sem):
            pltpu.make_async_copy(side_hbm.at[pl.ds(0, CB)], buf, sem).wait()

        def pair(t, carry):
            j0 = 2 * t
            wait_i(idx0, sem0)
            pltpu.sync_copy(onesb, acc.at[idx0], add=True)
            pltpu.async_copy(side_hbm.at[pl.ds(ioff(j0 + 2), CB)], idx0, sem0)
            wait_i(idx1, sem1)
            pltpu.sync_copy(onesb, acc.at[idx1], add=True)
            pltpu.async_copy(side_hbm.at[pl.ds(ioff(j0 + 3), CB)], idx1, sem1)
            return carry

        lax.fori_loop(0, NCH // 2, pair, 0)
        wait_i(idx0, sem0)
        wait_i(idx1, sem1)
        pltpu.sync_copy(side_hbm.at[pl.ds(base + NCH * CB, TAIL)], tidx)
        pltpu.sync_copy(onesb.at[pl.ds(0, TAIL)], acc.at[tidx], add=True)

    @pl.when(c == 0)
    def _():
        run_side(uidx_hbm)

    @pl.when(c == 1)
    def _():
        run_side(midx_hbm)

    plsc.subcore_barrier()
    eps = jnp.full((16,), 1e-9, jnp.float32)

    def fchunk(i, carry):
        r = (s + i * NS) * DFCH
        pltpu.sync_copy(acc.at[pl.ds(r, DFCH)], stage)

        def nrow(j, cc):
            stage2[j, pl.ds(0, 16)] = _rsqrt16(stage[j, pl.ds(0, 16)] + eps)
            return cc

        lax.fori_loop(0, DFCH, nrow, 0)
        pltpu.sync_copy(stage2, dtab_hbm.at[pl.ds(c * NU + r, DFCH)])
        return carry

    lax.fori_loop(0, nfl, fchunk, 0)


@functools.partial(
    pl.kernel,
    out_type=(jax.ShapeDtypeStruct((NC * NB_OUT, DH), jnp.float32),
              jax.ShapeDtypeStruct((NC * NT, DH), jnp.float32),
              jax.ShapeDtypeStruct((NC * NT, DH), jnp.float32),
              jax.ShapeDtypeStruct((NC * NT, DH), jnp.float32),
              jax.ShapeDtypeStruct((NC * NT, DH), jnp.float32)),
    mesh=_mesh,
    compiler_params=_sc_params,
    scratch_types=[
        pltpu.VMEM((CB,), jnp.int32),
        pltpu.VMEM((CB,), jnp.int32),
        pltpu.VMEM((CB,), jnp.int32),
        pltpu.VMEM((CB,), jnp.int32),
        pltpu.VMEM((CB, DH), jnp.float32),
        pltpu.VMEM((CB, DH), jnp.float32),
        pltpu.VMEM((TAIL,), jnp.int32),
        pltpu.VMEM((TAIL,), jnp.int32),
        pltpu.VMEM((TAIL, DH), jnp.float32),
        pltpu.VMEM((FCH, DH), jnp.float32),
        pltpu.VMEM((FCH, DH), jnp.float32),
        pltpu.VMEM((FCH, 16), jnp.float32),
        pltpu.VMEM((GCH,), jnp.int32),
        pltpu.VMEM((GCH,), jnp.int32),
        pltpu.VMEM((GCH, DH), jnp.float32),
        pltpu.VMEM((GCH, DH), jnp.float32),
        pltpu.VMEM((GCH, 16), jnp.float32),
        pltpu.VMEM((GCH, DH), jnp.float32),
        pltpu.VMEM_SHARED((NU, DH), jnp.float32),
        pltpu.SemaphoreType.DMA,
        pltpu.SemaphoreType.DMA,
        pltpu.SemaphoreType.DMA,
        pltpu.SemaphoreType.DMA,
    ],
)
def _mega_kernel(e0h_hbm, dtab_hbm, gsrca_hbm, gsrcb_hbm, dsta_hbm, dstb_hbm,
                 lidx2_hbm, lidx_hbm,
                 out_hbm, s1_hbm, s2_hbm, s3_hbm, t3_hbm,
                 sidx0, sidx1, didx0, didx1, rows0, rows1,
                 tsidx, tdidx, trows, zbuf, fbuf, dbuf,
                 lib, lrib, ls3, lt3, ldt, lout, acc,
                 semi0, semi1, semg0, semg1):
    c = lax.axis_index("c")
    s = lax.axis_index("s")

    _zero_fill(zbuf, FCH, DH)

    nfl = 39 + (s < 1).astype(jnp.int32)
    nfl2 = 78 + (s < 2).astype(jnp.int32)

    def zero_acc():
        def zchunk(i, carry):
            pltpu.sync_copy(zbuf, acc.at[pl.ds((s + i * NS) * FCH, FCH)])
            return carry

        lax.fori_loop(0, nfl, zchunk, 0)

    def prep(i, carry):
        r = (s + i * NS) * FCH
        pltpu.sync_copy(e0h_hbm.at[pl.ds(c * NT + r, FCH)], fbuf)
        pltpu.sync_copy(dtab_hbm.at[pl.ds(r, FCH)], dbuf)

        def srow(j, cc):
            d = dbuf[j, pl.ds(0, 16)]
            fbuf[j, pl.ds(0, 16)] = fbuf[j, pl.ds(0, 16)] * d
            fbuf[j, pl.ds(16, 16)] = fbuf[j, pl.ds(16, 16)] * d
            return cc

        lax.fori_loop(0, FCH, srow, 0)
        pltpu.sync_copy(fbuf, s1_hbm.at[pl.ds(c * NT + r, FCH)])
        return carry

    lax.fori_loop(0, nfl2, prep, 0)
    zero_acc()
    plsc.subcore_barrier()

    def phase(src_hbm, dst_hbm, table_hbm):
        base_s = c * NE + s * EPT
        base_d = s * EPT

        def soff(j):
            return c * NE + jnp.minimum(s * EPT + j * CB, NE - CB)

        def doff(j):
            return jnp.minimum(base_d + j * CB, NE - CB)

        def issue_idx(j, si, di, sem):
            pltpu.async_copy(src_hbm.at[pl.ds(soff(j), CB)], si, sem)
            pltpu.async_copy(dst_hbm.at[pl.ds(doff(j), CB)], di, sem)

        def wait_idx(si, di, sem):
            pltpu.make_async_copy(src_hbm.at[pl.ds(0, CB)], si, sem).wait()
            pltpu.make_async_copy(dst_hbm.at[pl.ds(0, CB)], di, sem).wait()

        def wait_gather(si, rw, sem):
            pltpu.make_async_copy(table_hbm.at[si], rw, sem).wait()

        issue_idx(0, sidx0, didx0, semi0)
        issue_idx(1, sidx1, didx1, semi1)
        wait_idx(sidx0, didx0, semi0)
        pltpu.async_copy(table_hbm.at[sidx0], rows0, semg0)

        def pair(t, carry):
            j0 = 2 * t
            wait_idx(sidx1, didx1, semi1)
            pltpu.async_copy(table_hbm.at[sidx1], rows1, semg1)
            wait_gather(sidx0, rows0, semg0)
            pltpu.sync_copy(rows0, acc.at[didx0], add=True)
            issue_idx(j0 + 2, sidx0, didx0, semi0)
            wait_idx(sidx0, didx0, semi0)
            pltpu.async_copy(table_hbm.at[sidx0], rows0, semg0)
            wait_gather(sidx1, rows1, semg1)
            pltpu.sync_copy(rows1, acc.at[didx1], add=True)
            issue_idx(j0 + 3, sidx1, didx1, semi1)
            return carry

        lax.fori_loop(0, NCH // 2, pair, 0)
        wait_gather(sidx0, rows0, semg0)
        wait_idx(sidx1, didx1, semi1)
        pltpu.sync_copy(src_hbm.at[pl.ds(base_s + NCH * CB, TAIL)], tsidx)
        pltpu.sync_copy(dst_hbm.at[pl.ds(base_d + NCH * CB, TAIL)], tdidx)
        pltpu.async_copy(table_hbm.at[tsidx], trows, semg0).wait()
        pltpu.sync_copy(trows, acc.at[tdidx], add=True)

    def flush(table_out, row_off, raw):
        def fchunk(i, carry):
            r = (s + i * NS) * FCH
            pltpu.sync_copy(acc.at[pl.ds(r, FCH)], fbuf)
            pltpu.sync_copy(zbuf, acc.at[pl.ds(r, FCH)])
            if not raw:
                pltpu.sync_copy(dtab_hbm.at[pl.ds(row_off + r, FCH)], dbuf)

                def srow(j, cc):
                    d = dbuf[j, pl.ds(0, 16)]
                    d2 = d * d
                    fbuf[j, pl.ds(0, 16)] = fbuf[j, pl.ds(0, 16)] * d2
                    fbuf[j, pl.ds(16, 16)] = fbuf[j, pl.ds(16, 16)] * d2
                    return cc

                lax.fori_loop(0, FCH, srow, 0)
            pltpu.sync_copy(fbuf,
                            table_out.at[pl.ds(c * NT + row_off + r, FCH)])
            return carry

        lax.fori_loop(0, nfl, fchunk, 0)

    def layer(table_in, table_out, raw):
        phase(gsrca_hbm, dsta_hbm, table_in)
        plsc.subcore_barrier()
        flush(table_out, 0, raw)
        plsc.subcore_barrier()
        phase(gsrcb_hbm, dstb_hbm, table_in)
        plsc.subcore_barrier()
        flush(table_out, NU, raw)
        plsc.subcore_barrier()

    layer(s1_hbm, s2_hbm, False)
    layer(s2_hbm, s3_hbm, False)
    layer(s3_hbm, t3_hbm, True)

    q = jnp.full((16,), 0.25, jnp.float32)

    le0 = rows0.at[pl.ds(0, GCH)]
    ls2 = rows1.at[pl.ds(0, GCH)]

    def lchunk(t, carry):
        off = s * RPT + t * GCH
        pltpu.sync_copy(lidx2_hbm.at[pl.ds(c * NB_OUT + off, GCH)], lib)
        pltpu.sync_copy(lidx_hbm.at[pl.ds(off, GCH)], lrib)
        pltpu.async_copy(e0h_hbm.at[lib], le0, semg0)
        pltpu.async_copy(s2_hbm.at[lib], ls2, semg0)
        pltpu.async_copy(s3_hbm.at[lib], ls3, semg0)
        pltpu.async_copy(t3_hbm.at[lib], lt3, semg0)
        pltpu.async_copy(dtab_hbm.at[lrib], ldt, semg1)
        for _ in range(4):
            pltpu.make_async_copy(e0h_hbm.at[lib], le0, semg0).wait()
        pltpu.make_async_copy(dtab_hbm.at[lrib], ldt, semg1).wait()

        def lrow(j, cc):
            d = ldt[j, pl.ds(0, 16)]
            for g in (0, 16):
                acc_g = (rows0[j, pl.ds(g, 16)]
                         + (rows1[j, pl.ds(g, 16)] + ls3[j, pl.ds(g, 16)]) / d
                         + d * lt3[j, pl.ds(g, 16)])
                lout[j, pl.ds(g, 16)] = acc_g * q
            return cc

        lax.fori_loop(0, GCH, lrow, 0)
        pltpu.sync_copy(lout, out_hbm.at[pl.ds(c * NB_OUT + off, GCH)])
        return carry

    lax.fori_loop(0, NGC, lchunk, 0)


def kernel(E0, users, pos_movies, neg_movies, user_index, movie_index):
    dtab = _degree_kernel(user_index, movie_index)

    e0h = jnp.concatenate([E0[:, :DH], E0[:, DH:]], axis=0)
    movie_plus = movie_index + NU
    gsrca = jnp.concatenate([movie_plus, movie_plus + NT])
    gsrcb = jnp.concatenate([user_index, user_index + NT])
    all_idx = jnp.concatenate([users, pos_movies + NU, neg_movies + NU])
    lidx2 = jnp.concatenate([all_idx, all_idx + NT])

    out2, _, _, _, _ = _mega_kernel(e0h, dtab, gsrca, gsrcb,
                                    user_index, movie_index, lidx2, all_idx)
    full = jnp.concatenate([out2[:NB_OUT], out2[NB_OUT:]], axis=1)
    usr = full[:4096]
    pos = full[4096:8192]
    neg = full[8192:]
    return (usr, pos, neg)


# speedup vs baseline: 22.6563x; 1.2665x over previous
import functools

import jax
import jax.numpy as jnp
from jax import lax
from jax.experimental import pallas as pl
from jax.experimental.pallas import tpu as pltpu
from jax.experimental.pallas import tpu_sc as plsc

NU = 50000
NM = 50000
NT = NU + NM
NE = 800000
DIM = 64
DH = 32
NC = 2
NS = 16
CB = 128
EPT = NE // NS
NCH = EPT // CB
TAIL = EPT - NCH * CB
DFCH = 200
FCH = 80
NFC = NU // FCH
NPC = NT // FCH
NB_OUT = 4096 + 4096 + 4 * 4096
RPT = NB_OUT // NS
GCH = 64
NGC = RPT // GCH

_mesh = plsc.VectorSubcoreMesh(core_axis_name="c", subcore_axis_name="s",
                               num_cores=NC, num_subcores=NS)
_sc_params = pltpu.CompilerParams(use_tc_tiling_on_sc=False)


def _zero_fill(buf, nrows, ncols):
    z = jnp.zeros((16,), jnp.float32)

    def body(i, carry):
        for c0 in range(0, ncols, 16):
            buf[i, pl.ds(c0, 16)] = z
        return carry

    lax.fori_loop(0, nrows, body, 0)


def _rsqrt16(x):
    magic = jnp.full((16,), 0x5F3759DF, jnp.int32)
    one = jnp.full((16,), 1, jnp.int32)
    c15 = jnp.full((16,), 1.5, jnp.float32)
    xi = lax.bitcast_convert_type(x, jnp.int32)
    y = lax.bitcast_convert_type(magic - lax.shift_right_logical(xi, one),
                                 jnp.float32)
    nhx = x * jnp.full((16,), -0.5, jnp.float32)
    for _ in range(4):
        y = y * (c15 + nhx * y * y)
    return y


@functools.partial(
    pl.kernel,
    out_type=jax.ShapeDtypeStruct((NT, 16), jnp.float32),
    mesh=_mesh,
    compiler_params=_sc_params,
    scratch_types=[
        pltpu.VMEM((CB,), jnp.int32),
        pltpu.VMEM((CB,), jnp.int32),
        pltpu.VMEM((TAIL,), jnp.int32),
        pltpu.VMEM((CB, 16), jnp.float32),
        pltpu.VMEM((DFCH, 16), jnp.float32),
        pltpu.VMEM((DFCH, 16), jnp.float32),
        pltpu.VMEM_SHARED((NU, 16), jnp.float32),
        pltpu.SemaphoreType.DMA,
        pltpu.SemaphoreType.DMA,
    ],
)
def _degree_kernel(uidx_hbm, midx_hbm, dtab_hbm,
                   idx0, idx1, tidx, onesb, stage, stage2, acc, sem0, sem1):
    c = lax.axis_index("c")
    s = lax.axis_index("s")

    one = jnp.ones((16,), jnp.float32)

    def ones_body(i, carry):
        onesb[i, pl.ds(0, 16)] = one
        return carry

    lax.fori_loop(0, CB, ones_body, 0)
    _zero_fill(stage, DFCH, 16)

    nfl = 15 + (s < 10).astype(jnp.int32)

    def zchunk(i, carry):
        pltpu.sync_copy(stage, acc.at[pl.ds((s + i * NS) * DFCH, DFCH)])
        return carry

    lax.fori_loop(0, nfl, zchunk, 0)
    plsc.subcore_barrier()

    def run_side(side_hbm):
        base = s * EPT

        def ioff(j):
            return jnp.minimum(base + j * CB, NE - CB)

        pltpu.async_copy(side_hbm.at[pl.ds(ioff(0), CB)], idx0, sem0)
        pltpu.async_copy(side_hbm.at[pl.ds(ioff(1), CB)], idx1, sem1)

        def wait_i(buf, sem):
            pltpu.make_async_copy(side_hbm.at[pl.ds(0, CB)], buf, sem).wait()

        def pair(t, carry):
            j0 = 2 * t
            wait_i(idx0, sem0)
            pltpu.sync_copy(onesb, acc.at[idx0], add=True)
            pltpu.async_copy(side_hbm.at[pl.ds(ioff(j0 + 2), CB)], idx0, sem0)
            wait_i(idx1, sem1)
            pltpu.sync_copy(onesb, acc.at[idx1], add=True)
            pltpu.async_copy(side_hbm.at[pl.ds(ioff(j0 + 3), CB)], idx1, sem1)
            return carry

        lax.fori_loop(0, NCH // 2, pair, 0)
        wait_i(idx0, sem0)
        wait_i(idx1, sem1)
        pltpu.sync_copy(side_hbm.at[pl.ds(base + NCH * CB, TAIL)], tidx)
        pltpu.sync_copy(onesb.at[pl.ds(0, TAIL)], acc.at[tidx], add=True)

    @pl.when(c == 0)
    def _():
        run_side(uidx_hbm)

    @pl.when(c == 1)
    def _():
        run_side(midx_hbm)

    plsc.subcore_barrier()
    eps = jnp.full((16,), 1e-9, jnp.float32)

    def fchunk(i, carry):
        r = (s + i * NS) * DFCH
        pltpu.sync_copy(acc.at[pl.ds(r, DFCH)], stage)

        def nrow(j, cc):
            stage2[j, pl.ds(0, 16)] = _rsqrt16(stage[j, pl.ds(0, 16)] + eps)
            return cc

        lax.fori_loop(0, DFCH, nrow, 0)
        pltpu.sync_copy(stage2, dtab_hbm.at[pl.ds(c * NU + r, DFCH)])
        return carry

    lax.fori_loop(0, nfl, fchunk, 0)


@functools.partial(
    pl.kernel,
    out_type=(jax.ShapeDtypeStruct((NC * NB_OUT, DH), jnp.float32),
              jax.ShapeDtypeStruct((NC * NT, DH), jnp.float32),
              jax.ShapeDtypeStruct((NC * NT, DH), jnp.float32),
              jax.ShapeDtypeStruct((NC * NT, DH), jnp.float32),
              jax.ShapeDtypeStruct((NC * NT, DH), jnp.float32)),
    mesh=_mesh,
    compiler_params=_sc_params,
    scratch_types=[
        pltpu.VMEM((CB,), jnp.int32),
        pltpu.VMEM((CB,), jnp.int32),
        pltpu.VMEM((CB,), jnp.int32),
        pltpu.VMEM((CB,), jnp.int32),
        pltpu.VMEM((CB,), jnp.int32),
        pltpu.VMEM((CB,), jnp.int32),
        pltpu.VMEM((CB, DH), jnp.float32),
        pltpu.VMEM((CB, DH), jnp.float32),
        pltpu.VMEM((CB, DH), jnp.float32),
        pltpu.VMEM((TAIL,), jnp.int32),
        pltpu.VMEM((TAIL,), jnp.int32),
        pltpu.VMEM((TAIL, DH), jnp.float32),
        pltpu.VMEM((FCH, DH), jnp.float32),
        pltpu.VMEM((FCH, DH), jnp.float32),
        pltpu.VMEM((FCH, 16), jnp.float32),
        pltpu.VMEM((GCH,), jnp.int32),
        pltpu.VMEM((GCH,), jnp.int32),
        pltpu.VMEM((GCH, DH), jnp.float32),
        pltpu.VMEM((GCH, DH), jnp.float32),
        pltpu.VMEM((GCH, 16), jnp.float32),
        pltpu.VMEM((GCH, DH), jnp.float32),
        pltpu.VMEM_SHARED((NU, DH), jnp.float32),
        pltpu.SemaphoreType.DMA,
        pltpu.SemaphoreType.DMA,
        pltpu.SemaphoreType.DMA,
        pltpu.SemaphoreType.DMA,
        pltpu.SemaphoreType.DMA,
        pltpu.SemaphoreType.DMA,
        pltpu.SemaphoreType.DMA,
        pltpu.SemaphoreType.DMA,
        pltpu.SemaphoreType.DMA,
    ],
)
def _mega_kernel(e0h_hbm, dtab_hbm, gsrca_hbm, gsrcb_hbm, dsta_hbm, dstb_hbm,
                 lidx2_hbm, lidx_hbm,
                 out_hbm, s1_hbm, s2_hbm, s3_hbm, t3_hbm,
                 sidx0, sidx1, sidx2, didx0, didx1, didx2,
                 rows0, rows1, rows2,
                 tsidx, tdidx, trows, zbuf, fbuf, dbuf,
                 lib, lrib, ls3, lt3, ldt, lout, acc,
                 semi0, semi1, semi2,
                 semg0, semg1, semg2,
                 sems0, sems1, sems2):
    c = lax.axis_index("c")
    s = lax.axis_index("s")

    _zero_fill(zbuf, FCH, DH)

    nfl = 39 + (s < 1).astype(jnp.int32)
    nfl2 = 78 + (s < 2).astype(jnp.int32)

    def zero_acc():
        def zchunk(i, carry):
            pltpu.sync_copy(zbuf, acc.at[pl.ds((s + i * NS) * FCH, FCH)])
            return carry

        lax.fori_loop(0, nfl, zchunk, 0)

    def prep(i, carry):
        r = (s + i * NS) * FCH
        pltpu.sync_copy(e0h_hbm.at[pl.ds(c * NT + r, FCH)], fbuf)
        pltpu.sync_copy(dtab_hbm.at[pl.ds(r, FCH)], dbuf)

        def srow(j, cc):
            d = dbuf[j, pl.ds(0, 16)]
            fbuf[j, pl.ds(0, 16)] = fbuf[j, pl.ds(0, 16)] * d
            fbuf[j, pl.ds(16, 16)] = fbuf[j, pl.ds(16, 16)] * d
            return cc

        lax.fori_loop(0, FCH, srow, 0)
        pltpu.sync_copy(fbuf, s1_hbm.at[pl.ds(c * NT + r, FCH)])
        return carry

    lax.fori_loop(0, nfl2, prep, 0)
    zero_acc()
    plsc.subcore_barrier()

    slots = ((sidx0, didx0, rows0, semi0, semg0, sems0),
             (sidx1, didx1, rows1, semi1, semg1, sems1),
             (sidx2, didx2, rows2, semi2, semg2, sems2))

    def phase(src_hbm, dst_hbm, table_hbm):
        base_s = c * NE + s * EPT
        base_d = s * EPT

        def soff(j):
            return c * NE + jnp.minimum(s * EPT + j * CB, NE - CB)

        def doff(j):
            return jnp.minimum(base_d + j * CB, NE - CB)

        def issue_idx(j, k):
            si, di, _, semi, _, _ = slots[k]
            pltpu.async_copy(src_hbm.at[pl.ds(soff(j), CB)], si, semi)
            pltpu.async_copy(dst_hbm.at[pl.ds(doff(j), CB)], di, semi)

        def wait_idx(k):
            si, di, _, semi, _, _ = slots[k]
            pltpu.make_async_copy(src_hbm.at[pl.ds(0, CB)], si, semi).wait()
            pltpu.make_async_copy(dst_hbm.at[pl.ds(0, CB)], di, semi).wait()

        def wait_scatter(k):
            _, di, rw, _, _, sems = slots[k]
            pltpu.make_async_copy(rw, acc.at[di], sems).wait()

        issue_idx(0, 0)

        def group(g, carry):
            for u in range(3):
                t = 3 * g + u
                k0 = u
                k1 = (u + 1) % 3
                k2 = (u + 2) % 3

                @pl.when(jnp.logical_and(t >= 2, t <= NCH + 1))
                def _():
                    wait_scatter(k1)

                @pl.when(t <= NCH - 2)
                def _():
                    issue_idx(t + 1, k1)

                @pl.when(t <= NCH - 1)
                def _():
                    wait_idx(k0)
                    si, _, rw, _, semg, _ = slots[k0]
                    pltpu.async_copy(table_hbm.at[si], rw, semg)

                @pl.when(jnp.logical_and(t >= 1, t <= NCH))
                def _():
                    si, di, rw, _, semg, sems = slots[k2]
                    pltpu.make_async_copy(table_hbm.at[si], rw, semg).wait()
                    pltpu.async_copy(rw, acc.at[di], sems, add=True)
            return carry

        lax.fori_loop(0, (NCH + 3) // 3, group, 0)
        pltpu.sync_copy(src_hbm.at[pl.ds(base_s + NCH * CB, TAIL)], tsidx)
        pltpu.sync_copy(dst_hbm.at[pl.ds(base_d + NCH * CB, TAIL)], tdidx)
        pltpu.async_copy(table_hbm.at[tsidx], trows, semg0).wait()
        pltpu.sync_copy(trows, acc.at[tdidx], add=True)

    def flush(table_out, row_off, raw):
        def fchunk(i, carry):
            r = (s + i * NS) * FCH
            pltpu.sync_copy(acc.at[pl.ds(r, FCH)], fbuf)
            pltpu.sync_copy(zbuf, acc.at[pl.ds(r, FCH)])
            if not raw:
                pltpu.sync_copy(dtab_hbm.at[pl.ds(row_off + r, FCH)], dbuf)

                def srow(j, cc):
                    d = dbuf[j, pl.ds(0, 16)]
                    d2 = d * d
                    fbuf[j, pl.ds(0, 16)] = fbuf[j, pl.ds(0, 16)] * d2
                    fbuf[j, pl.ds(16, 16)] = fbuf[j, pl.ds(16, 16)] * d2
                    return cc

                lax.fori_loop(0, FCH, srow, 0)
            pltpu.sync_copy(fbuf,
                            table_out.at[pl.ds(c * NT + row_off + r, FCH)])
            return carry

        lax.fori_loop(0, nfl, fchunk, 0)

    def layer(table_in, table_out, raw):
        phase(gsrca_hbm, dsta_hbm, table_in)
        plsc.subcore_barrier()
        flush(table_out, 0, raw)
        plsc.subcore_barrier()
        phase(gsrcb_hbm, dstb_hbm, table_in)
        plsc.subcore_barrier()
        flush(table_out, NU, raw)
        plsc.subcore_barrier()

    layer(s1_hbm, s2_hbm, False)
    layer(s2_hbm, s3_hbm, False)
    layer(s3_hbm, t3_hbm, True)

    q = jnp.full((16,), 0.25, jnp.float32)

    le0 = rows0.at[pl.ds(0, GCH)]
    ls2 = rows1.at[pl.ds(0, GCH)]

    def lchunk(t, carry):
        off = s * RPT + t * GCH
        pltpu.sync_copy(lidx2_hbm.at[pl.ds(c * NB_OUT + off, GCH)], lib)
        pltpu.sync_copy(lidx_hbm.at[pl.ds(off, GCH)], lrib)
        pltpu.async_copy(e0h_hbm.at[lib], le0, semg0)
        pltpu.async_copy(s2_hbm.at[lib], ls2, semg0)
        pltpu.async_copy(s3_hbm.at[lib], ls3, semg0)
        pltpu.async_copy(t3_hbm.at[lib], lt3, semg0)
        pltpu.async_copy(dtab_hbm.at[lrib], ldt, semg1)
        for _ in range(4):
            pltpu.make_async_copy(e0h_hbm.at[lib], le0, semg0).wait()
        pltpu.make_async_copy(dtab_hbm.at[lrib], ldt, semg1).wait()

        def lrow(j, cc):
            d = ldt[j, pl.ds(0, 16)]
            for g in (0, 16):
                acc_g = (rows0[j, pl.ds(g, 16)]
                         + (rows1[j, pl.ds(g, 16)] + ls3[j, pl.ds(g, 16)]) / d
                         + d * lt3[j, pl.ds(g, 16)])
                lout[j, pl.ds(g, 16)] = acc_g * q
            return cc

        lax.fori_loop(0, GCH, lrow, 0)
        pltpu.sync_copy(lout, out_hbm.at[pl.ds(c * NB_OUT + off, GCH)])
        return carry

    lax.fori_loop(0, NGC, lchunk, 0)


def kernel(E0, users, pos_movies, neg_movies, user_index, movie_index):
    dtab = _degree_kernel(user_index, movie_index)

    e0h = jnp.concatenate([E0[:, :DH], E0[:, DH:]], axis=0)
    movie_plus = movie_index + NU
    gsrca = jnp.concatenate([movie_plus, movie_plus + NT])
    gsrcb = jnp.concatenate([user_index, user_index + NT])
    all_idx = jnp.concatenate([users, pos_movies + NU, neg_movies + NU])
    lidx2 = jnp.concatenate([all_idx, all_idx + NT])

    out2, _, _, _, _ = _mega_kernel(e0h, dtab, gsrca, gsrcb,
                                    user_index, movie_index, lidx2, all_idx)
    full = jnp.concatenate([out2[:NB_OUT], out2[NB_OUT:]], axis=1)
    usr = full[:4096]
    pos = full[4096:8192]
    neg = full[8192:]
    return (usr, pos, neg)


# speedup vs baseline: 26.7969x; 1.1828x over previous
import functools

import jax
import jax.numpy as jnp
from jax import lax
from jax.experimental import pallas as pl
from jax.experimental.pallas import tpu as pltpu
from jax.experimental.pallas import tpu_sc as plsc

NU = 50000
NM = 50000
NT = NU + NM
NE = 800000
DIM = 64
DH = 32
NC = 2
NS = 16
CB = 128
EPT = NE // NS
NCH = EPT // CB
TAIL = EPT - NCH * CB
DFCH = 200
FCH = 80
NFC = NU // FCH
NPC = NT // FCH
NB_OUT = 4096 + 4096 + 4 * 4096
RPT = NB_OUT // NS
GCH = 64
NGC = RPT // GCH

_mesh = plsc.VectorSubcoreMesh(core_axis_name="c", subcore_axis_name="s",
                               num_cores=NC, num_subcores=NS)
_sc_params = pltpu.CompilerParams(use_tc_tiling_on_sc=False)


def _zero_fill(buf, nrows, ncols):
    z = jnp.zeros((16,), jnp.float32)

    def body(i, carry):
        for c0 in range(0, ncols, 16):
            buf[i, pl.ds(c0, 16)] = z
        return carry

    lax.fori_loop(0, nrows, body, 0)


def _rsqrt16(x):
    magic = jnp.full((16,), 0x5F3759DF, jnp.int32)
    one = jnp.full((16,), 1, jnp.int32)
    c15 = jnp.full((16,), 1.5, jnp.float32)
    xi = lax.bitcast_convert_type(x, jnp.int32)
    y = lax.bitcast_convert_type(magic - lax.shift_right_logical(xi, one),
                                 jnp.float32)
    nhx = x * jnp.full((16,), -0.5, jnp.float32)
    for _ in range(4):
        y = y * (c15 + nhx * y * y)
    return y


@functools.partial(
    pl.kernel,
    out_type=jax.ShapeDtypeStruct((NT, 16), jnp.float32),
    mesh=_mesh,
    compiler_params=_sc_params,
    scratch_types=[
        pltpu.VMEM((CB,), jnp.int32),
        pltpu.VMEM((CB,), jnp.int32),
        pltpu.VMEM((TAIL,), jnp.int32),
        pltpu.VMEM((CB, 16), jnp.float32),
        pltpu.VMEM((DFCH, 16), jnp.float32),
        pltpu.VMEM((DFCH, 16), jnp.float32),
        pltpu.VMEM_SHARED((NU, 16), jnp.float32),
        pltpu.SemaphoreType.DMA,
        pltpu.SemaphoreType.DMA,
    ],
)
def _degree_kernel(uidx_hbm, midx_hbm, dtab_hbm,
                   idx0, idx1, tidx, onesb, stage, stage2, acc, sem0, sem1):
    c = lax.axis_index("c")
    s = lax.axis_index("s")

    one = jnp.ones((16,), jnp.float32)

    def ones_body(i, carry):
        onesb[i, pl.ds(0, 16)] = one
        return carry

    lax.fori_loop(0, CB, ones_body, 0)
    _zero_fill(stage, DFCH, 16)

    nfl = 15 + (s < 10).astype(jnp.int32)

    def zchunk(i, carry):
        pltpu.sync_copy(stage, acc.at[pl.ds((s + i * NS) * DFCH, DFCH)])
        return carry

    lax.fori_loop(0, nfl, zchunk, 0)
    plsc.subcore_barrier()

    def run_side(side_hbm):
        base = s * EPT

        def ioff(j):
            return jnp.minimum(base + j * CB, NE - CB)

        pltpu.async_copy(side_hbm.at[pl.ds(ioff(0), CB)], idx0, sem0)
        pltpu.async_copy(side_hbm.at[pl.ds(ioff(1), CB)], idx1, sem1)

        def wait_i(buf, sem):
            pltpu.make_async_copy(side_hbm.at[pl.ds(0, CB)], buf, sem).wait()

        def pair(t, carry):
            j0 = 2 * t
            wait_i(idx0, sem0)
            pltpu.sync_copy(onesb, acc.at[idx0], add=True)
            pltpu.async_copy(side_hbm.at[pl.ds(ioff(j0 + 2), CB)], idx0, sem0)
            wait_i(idx1, sem1)
            pltpu.sync_copy(onesb, acc.at[idx1], add=True)
            pltpu.async_copy(side_hbm.at[pl.ds(ioff(j0 + 3), CB)], idx1, sem1)
            return carry

        lax.fori_loop(0, NCH // 2, pair, 0)
        wait_i(idx0, sem0)
        wait_i(idx1, sem1)
        pltpu.sync_copy(side_hbm.at[pl.ds(base + NCH * CB, TAIL)], tidx)
        pltpu.sync_copy(onesb.at[pl.ds(0, TAIL)], acc.at[tidx], add=True)

    @pl.when(c == 0)
    def _():
        run_side(uidx_hbm)

    @pl.when(c == 1)
    def _():
        run_side(midx_hbm)

    plsc.subcore_barrier()
    eps = jnp.full((16,), 1e-9, jnp.float32)

    def fchunk(i, carry):
        r = (s + i * NS) * DFCH
        pltpu.sync_copy(acc.at[pl.ds(r, DFCH)], stage)

        def nrow(j, cc):
            stage2[j, pl.ds(0, 16)] = _rsqrt16(stage[j, pl.ds(0, 16)] + eps)
            return cc

        lax.fori_loop(0, DFCH, nrow, 0)
        pltpu.sync_copy(stage2, dtab_hbm.at[pl.ds(c * NU + r, DFCH)])
        return carry

    lax.fori_loop(0, nfl, fchunk, 0)


@functools.partial(
    pl.kernel,
    out_type=(jax.ShapeDtypeStruct((NC * NB_OUT, DH), jnp.float32),
              jax.ShapeDtypeStruct((NC * NT, DH), jnp.float32),
              jax.ShapeDtypeStruct((NC * NT, DH), jnp.float32),
              jax.ShapeDtypeStruct((NC * NT, DH), jnp.float32),
              jax.ShapeDtypeStruct((NC * NT, DH), jnp.float32)),
    mesh=_mesh,
    compiler_params=_sc_params,
    scratch_types=[
        pltpu.VMEM((CB,), jnp.int32),
        pltpu.VMEM((CB,), jnp.int32),
        pltpu.VMEM((CB,), jnp.int32),
        pltpu.VMEM((CB,), jnp.int32),
        pltpu.VMEM((CB,), jnp.int32),
        pltpu.VMEM((CB,), jnp.int32),
        pltpu.VMEM((CB,), jnp.int32),
        pltpu.VMEM((CB,), jnp.int32),
        pltpu.VMEM((CB,), jnp.int32),
        pltpu.VMEM((CB,), jnp.int32),
        pltpu.VMEM((CB, DH), jnp.float32),
        pltpu.VMEM((CB, DH), jnp.float32),
        pltpu.VMEM((CB, DH), jnp.float32),
        pltpu.VMEM((CB, DH), jnp.float32),
        pltpu.VMEM((CB, DH), jnp.float32),
        pltpu.VMEM((FCH, DH), jnp.float32),
        pltpu.VMEM((FCH, DH), jnp.float32),
        pltpu.VMEM((FCH, 16), jnp.float32),
        pltpu.VMEM((GCH,), jnp.int32),
        pltpu.VMEM((GCH,), jnp.int32),
        pltpu.VMEM((GCH, 16), jnp.float32),
        pltpu.VMEM_SHARED((NU, DH), jnp.float32),
        pltpu.SemaphoreType.DMA,
        pltpu.SemaphoreType.DMA,
        pltpu.SemaphoreType.DMA,
        pltpu.SemaphoreType.DMA,
        pltpu.SemaphoreType.DMA,
        pltpu.SemaphoreType.DMA,
        pltpu.SemaphoreType.DMA,
        pltpu.SemaphoreType.DMA,
        pltpu.SemaphoreType.DMA,
        pltpu.SemaphoreType.DMA,
        pltpu.SemaphoreType.DMA,
        pltpu.SemaphoreType.DMA,
        pltpu.SemaphoreType.DMA,
        pltpu.SemaphoreType.DMA,
        pltpu.SemaphoreType.DMA,
    ],
)
def _mega_kernel(e0h_hbm, dtab_hbm, gsrca_hbm, gsrcb_hbm, dsta_hbm, dstb_hbm,
                 lidx2_hbm, lidx_hbm,
                 out_hbm, s1_hbm, s2_hbm, s3_hbm, t3_hbm,
                 sidx0, sidx1, sidx2, sidx3, sidx4,
                 didx0, didx1, didx2, didx3, didx4,
                 rows0, rows1, rows2, rows3, rows4,
                 zbuf, fbuf, dbuf,
                 lib, lrib, ldt, acc,
                 semi0, semi1, semi2, semi3, semi4,
                 semg0, semg1, semg2, semg3, semg4,
                 sems0, sems1, sems2, sems3, sems4):
    c = lax.axis_index("c")
    s = lax.axis_index("s")

    _zero_fill(zbuf, FCH, DH)

    nfl = 39 + (s < 1).astype(jnp.int32)
    nfl2 = 78 + (s < 2).astype(jnp.int32)

    def zero_acc():
        def zchunk(i, carry):
            pltpu.sync_copy(zbuf, acc.at[pl.ds((s + i * NS) * FCH, FCH)])
            return carry

        lax.fori_loop(0, nfl, zchunk, 0)

    def prep(i, carry):
        r = (s + i * NS) * FCH
        pltpu.sync_copy(e0h_hbm.at[pl.ds(c * NT + r, FCH)], fbuf)
        pltpu.sync_copy(dtab_hbm.at[pl.ds(r, FCH)], dbuf)

        def srow(j, cc):
            d = dbuf[j, pl.ds(0, 16)]
            fbuf[j, pl.ds(0, 16)] = fbuf[j, pl.ds(0, 16)] * d
            fbuf[j, pl.ds(16, 16)] = fbuf[j, pl.ds(16, 16)] * d
            return cc

        lax.fori_loop(0, FCH, srow, 0)
        pltpu.sync_copy(fbuf, s1_hbm.at[pl.ds(c * NT + r, FCH)])
        return carry

    lax.fori_loop(0, nfl2, prep, 0)
    zero_acc()
    plsc.subcore_barrier()

    slots = ((sidx0, didx0, rows0, semi0, semg0, sems0),
             (sidx1, didx1, rows1, semi1, semg1, sems1),
             (sidx2, didx2, rows2, semi2, semg2, sems2),
             (sidx3, didx3, rows3, semi3, semg3, sems3),
             (sidx4, didx4, rows4, semi4, semg4, sems4))

    def phase(src_hbm, dst_hbm, table_hbm):
        base_s = c * NE + s * EPT
        base_d = s * EPT

        def soff(j):
            return c * NE + jnp.minimum(s * EPT + j * CB, NE - CB)

        def doff(j):
            return jnp.minimum(base_d + j * CB, NE - CB)

        def issue_idx(j, k):
            si, di, _, semi, _, _ = slots[k]
            pltpu.async_copy(src_hbm.at[pl.ds(soff(j), CB)], si, semi)
            pltpu.async_copy(dst_hbm.at[pl.ds(doff(j), CB)], di, semi)

        def wait_idx(k):
            si, di, _, semi, _, _ = slots[k]
            pltpu.make_async_copy(src_hbm.at[pl.ds(0, CB)], si, semi).wait()
            pltpu.make_async_copy(dst_hbm.at[pl.ds(0, CB)], di, semi).wait()

        def wait_scatter(k):
            _, di, rw, _, _, sems = slots[k]
            pltpu.make_async_copy(rw, acc.at[di], sems).wait()

        issue_idx(0, 0)

        def group(g, carry):
            for u in range(5):
                t = 5 * g + u
                k0 = u
                k1 = (u + 1) % 5
                k2 = (u + 2) % 5

                @pl.when(jnp.logical_and(t >= 4, t <= NCH + 3))
                def _():
                    wait_scatter(k1)

                @pl.when(t <= NCH - 2)
                def _():
                    issue_idx(t + 1, k1)

                @pl.when(t <= NCH - 1)
                def _():
                    wait_idx(k0)
                    si, _, rw, _, semg, _ = slots[k0]
                    pltpu.async_copy(table_hbm.at[si], rw, semg)

                @pl.when(jnp.logical_and(t >= 3, t <= NCH + 2))
                def _():
                    si, di, rw, _, semg, sems = slots[k2]
                    pltpu.make_async_copy(table_hbm.at[si], rw, semg).wait()
                    pltpu.async_copy(rw, acc.at[di], sems, add=True)
            return carry

        lax.fori_loop(0, (NCH + 5) // 5, group, 0)
        tsidx = sidx0.at[pl.ds(0, TAIL)]
        tdidx = didx0.at[pl.ds(0, TAIL)]
        trows = rows0.at[pl.ds(0, TAIL)]
        pltpu.sync_copy(src_hbm.at[pl.ds(base_s + NCH * CB, TAIL)], tsidx)
        pltpu.sync_copy(dst_hbm.at[pl.ds(base_d + NCH * CB, TAIL)], tdidx)
        pltpu.async_copy(table_hbm.at[tsidx], trows, semg0).wait()
        pltpu.sync_copy(trows, acc.at[tdidx], add=True)

    def flush(table_out, row_off, raw):
        def fchunk(i, carry):
            r = (s + i * NS) * FCH
            pltpu.sync_copy(acc.at[pl.ds(r, FCH)], fbuf)
            pltpu.sync_copy(zbuf, acc.at[pl.ds(r, FCH)])
            if not raw:
                pltpu.sync_copy(dtab_hbm.at[pl.ds(row_off + r, FCH)], dbuf)

                def srow(j, cc):
                    d = dbuf[j, pl.ds(0, 16)]
                    d2 = d * d
                    fbuf[j, pl.ds(0, 16)] = fbuf[j, pl.ds(0, 16)] * d2
                    fbuf[j, pl.ds(16, 16)] = fbuf[j, pl.ds(16, 16)] * d2
                    return cc

                lax.fori_loop(0, FCH, srow, 0)
            pltpu.sync_copy(fbuf,
                            table_out.at[pl.ds(c * NT + row_off + r, FCH)])
            return carry

        lax.fori_loop(0, nfl, fchunk, 0)

    def layer(table_in, table_out, raw):
        phase(gsrca_hbm, dsta_hbm, table_in)
        plsc.subcore_barrier()
        flush(table_out, 0, raw)
        plsc.subcore_barrier()
        phase(gsrcb_hbm, dstb_hbm, table_in)
        plsc.subcore_barrier()
        flush(table_out, NU, raw)
        plsc.subcore_barrier()

    layer(s1_hbm, s2_hbm, False)
    layer(s2_hbm, s3_hbm, False)
    layer(s3_hbm, t3_hbm, True)

    q = jnp.full((16,), 0.25, jnp.float32)

    le0 = rows0.at[pl.ds(0, GCH)]
    ls3 = rows0.at[pl.ds(GCH, GCH)]
    ls2 = rows1.at[pl.ds(0, GCH)]
    lt3 = rows1.at[pl.ds(GCH, GCH)]
    lout = rows2.at[pl.ds(0, GCH)]

    def lchunk(t, carry):
        off = s * RPT + t * GCH
        pltpu.sync_copy(lidx2_hbm.at[pl.ds(c * NB_OUT + off, GCH)], lib)
        pltpu.sync_copy(lidx_hbm.at[pl.ds(off, GCH)], lrib)
        pltpu.async_copy(e0h_hbm.at[lib], le0, semg0)
        pltpu.async_copy(s2_hbm.at[lib], ls2, semg0)
        pltpu.async_copy(s3_hbm.at[lib], ls3, semg0)
        pltpu.async_copy(t3_hbm.at[lib], lt3, semg0)
        pltpu.async_copy(dtab_hbm.at[lrib], ldt, semg1)
        for _ in range(4):
            pltpu.make_async_copy(e0h_hbm.at[lib], le0, semg0).wait()
        pltpu.make_async_copy(dtab_hbm.at[lrib], ldt, semg1).wait()

        def lrow(j, cc):
            d = ldt[j, pl.ds(0, 16)]
            for g in (0, 16):
                acc_g = (rows0[j, pl.ds(g, 16)]
                         + (rows1[j, pl.ds(g, 16)]
                            + rows0[GCH + j, pl.ds(g, 16)]) / d
                         + d * rows1[GCH + j, pl.ds(g, 16)])
                rows2[j, pl.ds(g, 16)] = acc_g * q
            return cc

        lax.fori_loop(0, GCH, lrow, 0)
        pltpu.sync_copy(lout, out_hbm.at[pl.ds(c * NB_OUT + off, GCH)])
        return carry

    lax.fori_loop(0, NGC, lchunk, 0)


def kernel(E0, users, pos_movies, neg_movies, user_index, movie_index):
    dtab = _degree_kernel(user_index, movie_index)

    e0h = jnp.concatenate([E0[:, :DH], E0[:, DH:]], axis=0)
    movie_plus = movie_index + NU
    gsrca = jnp.concatenate([movie_plus, movie_plus + NT])
    gsrcb = jnp.concatenate([user_index, user_index + NT])
    all_idx = jnp.concatenate([users, pos_movies + NU, neg_movies + NU])
    lidx2 = jnp.concatenate([all_idx, all_idx + NT])

    out2, _, _, _, _ = _mega_kernel(e0h, dtab, gsrca, gsrcb,
                                    user_index, movie_index, lidx2, all_idx)
    full = jnp.concatenate([out2[:NB_OUT], out2[NB_OUT:]], axis=1)
    usr = full[:4096]
    pos = full[4096:8192]
    neg = full[8192:]
    return (usr, pos, neg)
